# baseline (speedup 1.0000x reference)
import dataclasses

import jax
import jax.numpy as jnp
from jax import lax
from jax.experimental import pallas as pl
from jax.experimental.pallas import tpu as pltpu
from jax.experimental.pallas import tpu_sc as plsc

POOL = 7
SCALE = 0.0625
B, C, H, W = 2, 256, 38, 38
N = 1000
NEG = jnp.finfo(jnp.float32).min

WP = 40
HWP = H * WP
NCOMBO = 9
ZERO_ROW = NCOMBO * B * HWP

NBINS = N * POOL * POOL
NWORKERS = 32
CB = 32
BINS_PER_W = 1536
BINSP = NWORKERS * BINS_PER_W
NCHUNK = BINS_PER_W // CB


def _shift0(x, s, size):
    return jnp.concatenate([x[s:], jnp.broadcast_to(x[size - 1:], (s,) + x.shape[1:])], axis=0)


def _shift1(x, s, size):
    last = x[:, size - 1:]
    return jnp.concatenate([x[:, s:], jnp.broadcast_to(last, x.shape[:1] + (s,) + x.shape[2:])], axis=1)


def _pack_bf16_pair(x):
    u_lo = jax.lax.bitcast_convert_type(x[..., 0:C // 2], jnp.uint32)
    u_hi = jax.lax.bitcast_convert_type(x[..., C // 2:C], jnp.uint32)
    rne = lambda u: u + jnp.uint32(0x7FFF) + ((u >> 16) & jnp.uint32(1))
    lo16 = rne(u_lo) >> 16
    hi16 = rne(u_hi) & jnp.uint32(0xFFFF0000)
    return lo16 | hi16


def _table_body(fmap_ref, out_ref):
    for bb in range(B):
        f = fmap_ref[bb]
        th = f
        for kh in range(3):
            if kh:
                th = jnp.maximum(th, _shift0(th, 1 << (kh - 1), H))
            tw = th
            for kw in range(3):
                if kw:
                    tw = jnp.maximum(tw, _shift1(tw, 1 << (kw - 1), WP))
                out_ref[(kh * 3 + kw) * B + bb] = _pack_bf16_pair(tw)
    out_ref[NCOMBO * B] = jnp.zeros((H, WP, C // 2), jnp.uint32)


def _sc_body(table_hbm, idx_hbm, out_hbm, idx_v, rows_v, out_v, sem0, sem1):
    wid = lax.axis_index("s") * 2 + lax.axis_index("c")
    base = wid * BINS_PER_W

    def start_gather(t, buf, sem):
        cps = []
        for k in range(4):
            pltpu.sync_copy(idx_hbm.at[pl.ds(k * BINSP + base + t * CB, CB)],
                            idx_v.at[buf, k])
            cps.append(pltpu.async_copy(table_hbm.at[idx_v.at[buf, k]],
                                        rows_v.at[buf, k], sem))
        return cps

    def compute(buf):
        def bin_body(i):
            for j in range(C // 32):
                sl = pl.ds(j * 16, 16)
                r = [plsc.bitcast(rows_v[buf, k, i, sl], jnp.bfloat16)
                     for k in range(4)]
                m = jnp.maximum(jnp.maximum(r[0], r[1]), jnp.maximum(r[2], r[3]))
                out_v[i, sl] = plsc.bitcast(m, jnp.int32)
        pl.loop(0, CB)(bin_body)

    for cp in start_gather(0, 0, sem0):
        cp.wait()

    def chunk_body(t):
        @pl.when(t % 2 == 0)
        def _():
            cps = start_gather(t + 1, 1, sem1)
            compute(0)
            for cp in cps:
                cp.wait()

        @pl.when(t % 2 == 1)
        def _():
            cps = start_gather(t + 1, 0, sem0)
            compute(1)
            for cp in cps:
                cp.wait()

        pltpu.sync_copy(out_v, out_hbm.at[pl.ds(base + t * CB, CB)])

    pl.loop(0, NCHUNK - 1)(chunk_body)

    @pl.when((NCHUNK - 1) % 2 == 0)
    def _():
        compute(0)

    @pl.when((NCHUNK - 1) % 2 == 1)
    def _():
        compute(1)

    pltpu.sync_copy(out_v, out_hbm.at[pl.ds(base + (NCHUNK - 1) * CB, CB)])


def _unpack_body(in_ref, out_ref):
    x = in_ref[...]
    lo = jax.lax.bitcast_convert_type(x << 16, jnp.float32)
    hi = jax.lax.bitcast_convert_type(x & jnp.uint32(0xFFFF0000), jnp.float32)
    out_ref[0] = jnp.concatenate([lo, hi], axis=1)


def _roi_corner_idx(rois):
    b = rois[:, 0].astype(jnp.int32)
    rs_w = jnp.round(rois[:, 1] * SCALE).astype(jnp.int32)
    rs_h = jnp.round(rois[:, 2] * SCALE).astype(jnp.int32)
    re_w = jnp.round(rois[:, 3] * SCALE).astype(jnp.int32)
    re_h = jnp.round(rois[:, 4] * SCALE).astype(jnp.int32)
    roi_w = jnp.maximum(re_w - rs_w + 1, 1).astype(jnp.float32)
    roi_h = jnp.maximum(re_h - rs_h + 1, 1).astype(jnp.float32)
    bin_w = roi_w / POOL
    bin_h = roi_h / POOL
    p = jnp.arange(POOL, dtype=jnp.float32)[:, None]
    hstart = jnp.clip(jnp.floor(p * bin_h[None, :]).astype(jnp.int32) + rs_h[None, :], 0, H)
    hend = jnp.clip(jnp.ceil((p + 1.0) * bin_h[None, :]).astype(jnp.int32) + rs_h[None, :], 0, H)
    wstart = jnp.clip(jnp.floor(p * bin_w[None, :]).astype(jnp.int32) + rs_w[None, :], 0, W)
    wend = jnp.clip(jnp.ceil((p + 1.0) * bin_w[None, :]).astype(jnp.int32) + rs_w[None, :], 0, W)

    len_h = hend - hstart
    len_w = wend - wstart
    kh = (len_h >= 2).astype(jnp.int32) + (len_h >= 4).astype(jnp.int32)
    kw = (len_w >= 2).astype(jnp.int32) + (len_w >= 4).astype(jnp.int32)
    hA = jnp.clip(hstart, 0, H - 1)
    hB = jnp.clip(hend - (1 << kh), 0, H - 1)
    wA = jnp.clip(wstart, 0, W - 1)
    wB = jnp.clip(wend - (1 << kw), 0, W - 1)

    combo = (kh[:, None, :] * 3 + kw[None, :, :]) * B + b[None, None, :]
    cbase = combo * HWP
    valid = (len_h > 0)[:, None, :] & (len_w > 0)[None, :, :]
    pad = jnp.full((BINSP - NBINS,), ZERO_ROW, jnp.int32)

    def corner(h, w):
        flat = cbase + (h[:, None, :] * WP + w[None, :, :])
        flat = jnp.where(valid, flat, ZERO_ROW).reshape(NBINS)
        return jnp.concatenate([flat, pad])

    return jnp.concatenate(
        [corner(hA, wA), corner(hA, wB), corner(hB, wA), corner(hB, wB)])


def kernel(features, rois):
    fmap = jnp.pad(jnp.transpose(features, (0, 2, 3, 1)),
                   ((0, 0), (0, 0), (0, WP - W), (0, 0)))
    idx = _roi_corner_idx(rois)

    table_u32 = pl.pallas_call(
        _table_body,
        in_specs=[pl.BlockSpec((B, H, WP, C), lambda: (0, 0, 0, 0))],
        out_specs=pl.BlockSpec((NCOMBO * B + 1, H, WP, C // 2), lambda: (0, 0, 0, 0)),
        out_shape=jax.ShapeDtypeStruct((NCOMBO * B + 1, H, WP, C // 2), jnp.uint32),
    )(fmap).reshape((NCOMBO * B + 1) * HWP, C // 2)

    mesh = plsc.VectorSubcoreMesh(core_axis_name="c", subcore_axis_name="s")
    cp = pltpu.CompilerParams()
    if "needs_layout_passes" in pltpu.CompilerParams.__dataclass_fields__:
        cp = dataclasses.replace(cp, needs_layout_passes=False)
    sc_gather = pl.kernel(
        _sc_body,
        out_type=jax.ShapeDtypeStruct((BINSP, C // 2), jnp.uint32),
        mesh=mesh,
        compiler_params=cp,
        scratch_types=[
            pltpu.VMEM((2, 4, CB), jnp.int32),
            pltpu.VMEM((2, 4, CB, C // 2), jnp.uint32),
            pltpu.VMEM((CB, C // 2), jnp.uint32),
            pltpu.SemaphoreType.DMA,
            pltpu.SemaphoreType.DMA,
        ],
    )
    binmax = sc_gather(table_u32, idx)

    out = pl.pallas_call(
        _unpack_body,
        grid=(POOL * POOL,),
        in_specs=[pl.BlockSpec((N, C // 2), lambda s: (s, 0))],
        out_specs=pl.BlockSpec((1, N, C), lambda s: (s, 0, 0)),
        out_shape=jax.ShapeDtypeStruct((POOL * POOL, N, C), jnp.float32),
    )(binmax)
    return jnp.transpose(out.reshape(POOL, POOL, N, C), (2, 3, 0, 1))


# speedup vs baseline: 1.1908x; 1.1908x over previous
import dataclasses

import jax
import jax.numpy as jnp
from jax import lax
from jax.experimental import pallas as pl
from jax.experimental.pallas import tpu as pltpu
from jax.experimental.pallas import tpu_sc as plsc

POOL = 7
SCALE = 0.0625
B, C, H, W = 2, 256, 38, 38
N = 1000
NEG = jnp.finfo(jnp.float32).min

WP = 40
HWP = H * WP
NCOMBO = 9
ZERO_ROW = NCOMBO * B * HWP

NBINS = N * POOL * POOL
NWORKERS = 32
CB = 64
BINS_PER_W = 1536
BINSP = NWORKERS * BINS_PER_W
NCHUNK = BINS_PER_W // CB


def _shift0(x, s, size):
    return jnp.concatenate([x[s:], jnp.broadcast_to(x[size - 1:], (s,) + x.shape[1:])], axis=0)


def _shift1(x, s, size):
    last = x[:, size - 1:]
    return jnp.concatenate([x[:, s:], jnp.broadcast_to(last, x.shape[:1] + (s,) + x.shape[2:])], axis=1)


def _pack_bf16_pair(x):
    u_lo = jax.lax.bitcast_convert_type(x[..., 0:C // 2], jnp.uint32)
    u_hi = jax.lax.bitcast_convert_type(x[..., C // 2:C], jnp.uint32)
    rne = lambda u: u + jnp.uint32(0x7FFF) + ((u >> 16) & jnp.uint32(1))
    lo16 = rne(u_lo) >> 16
    hi16 = rne(u_hi) & jnp.uint32(0xFFFF0000)
    return lo16 | hi16


def _table_body(fmap_ref, out_ref):
    for bb in range(B):
        f = fmap_ref[bb]
        th = f
        for kh in range(3):
            if kh:
                th = jnp.maximum(th, _shift0(th, 1 << (kh - 1), H))
            tw = th
            for kw in range(3):
                if kw:
                    tw = jnp.maximum(tw, _shift1(tw, 1 << (kw - 1), WP))
                out_ref[(kh * 3 + kw) * B + bb] = _pack_bf16_pair(tw)
    out_ref[NCOMBO * B] = jnp.zeros((H, WP, C // 2), jnp.uint32)


def _sc_body(table_hbm, idx_hbm, out_hbm, idx_v, rows_v, out_v, sem0, sem1):
    wid = lax.axis_index("s") * 2 + lax.axis_index("c")
    base = wid * BINS_PER_W

    def start_gather(t, buf, sem):
        cps = []
        for k in range(4):
            pltpu.sync_copy(idx_hbm.at[pl.ds(k * BINSP + base + t * CB, CB)],
                            idx_v.at[buf, k])
            cps.append(pltpu.async_copy(table_hbm.at[idx_v.at[buf, k]],
                                        rows_v.at[buf, k], sem))
        return cps

    def compute(buf):
        def bin_body(i):
            for j in range(C // 32):
                sl = pl.ds(j * 16, 16)
                r = [plsc.bitcast(rows_v[buf, k, i, sl], jnp.bfloat16)
                     for k in range(4)]
                m = jnp.maximum(jnp.maximum(r[0], r[1]), jnp.maximum(r[2], r[3]))
                out_v[i, sl] = plsc.bitcast(m, jnp.int32)
        pl.loop(0, CB)(bin_body)

    for cp in start_gather(0, 0, sem0):
        cp.wait()

    def chunk_body(t):
        @pl.when(t % 2 == 0)
        def _():
            cps = start_gather(t + 1, 1, sem1)
            compute(0)
            for cp in cps:
                cp.wait()

        @pl.when(t % 2 == 1)
        def _():
            cps = start_gather(t + 1, 0, sem0)
            compute(1)
            for cp in cps:
                cp.wait()

        pltpu.sync_copy(out_v, out_hbm.at[pl.ds(base + t * CB, CB)])

    pl.loop(0, NCHUNK - 1)(chunk_body)

    @pl.when((NCHUNK - 1) % 2 == 0)
    def _():
        compute(0)

    @pl.when((NCHUNK - 1) % 2 == 1)
    def _():
        compute(1)

    pltpu.sync_copy(out_v, out_hbm.at[pl.ds(base + (NCHUNK - 1) * CB, CB)])


def _unpack_body(in_ref, out_ref):
    x = in_ref[...]
    lo = jax.lax.bitcast_convert_type(x << 16, jnp.float32)
    hi = jax.lax.bitcast_convert_type(x & jnp.uint32(0xFFFF0000), jnp.float32)
    out_ref[0] = jnp.concatenate([lo, hi], axis=1)


def _roi_corner_idx(rois):
    b = rois[:, 0].astype(jnp.int32)
    rs_w = jnp.round(rois[:, 1] * SCALE).astype(jnp.int32)
    rs_h = jnp.round(rois[:, 2] * SCALE).astype(jnp.int32)
    re_w = jnp.round(rois[:, 3] * SCALE).astype(jnp.int32)
    re_h = jnp.round(rois[:, 4] * SCALE).astype(jnp.int32)
    roi_w = jnp.maximum(re_w - rs_w + 1, 1).astype(jnp.float32)
    roi_h = jnp.maximum(re_h - rs_h + 1, 1).astype(jnp.float32)
    bin_w = roi_w / POOL
    bin_h = roi_h / POOL
    p = jnp.arange(POOL, dtype=jnp.float32)[:, None]
    hstart = jnp.clip(jnp.floor(p * bin_h[None, :]).astype(jnp.int32) + rs_h[None, :], 0, H)
    hend = jnp.clip(jnp.ceil((p + 1.0) * bin_h[None, :]).astype(jnp.int32) + rs_h[None, :], 0, H)
    wstart = jnp.clip(jnp.floor(p * bin_w[None, :]).astype(jnp.int32) + rs_w[None, :], 0, W)
    wend = jnp.clip(jnp.ceil((p + 1.0) * bin_w[None, :]).astype(jnp.int32) + rs_w[None, :], 0, W)

    len_h = hend - hstart
    len_w = wend - wstart
    kh = (len_h >= 2).astype(jnp.int32) + (len_h >= 4).astype(jnp.int32)
    kw = (len_w >= 2).astype(jnp.int32) + (len_w >= 4).astype(jnp.int32)
    hA = jnp.clip(hstart, 0, H - 1)
    hB = jnp.clip(hend - (1 << kh), 0, H - 1)
    wA = jnp.clip(wstart, 0, W - 1)
    wB = jnp.clip(wend - (1 << kw), 0, W - 1)

    combo = (kh[:, None, :] * 3 + kw[None, :, :]) * B + b[None, None, :]
    cbase = combo * HWP
    valid = (len_h > 0)[:, None, :] & (len_w > 0)[None, :, :]
    pad = jnp.full((BINSP - NBINS,), ZERO_ROW, jnp.int32)

    def corner(h, w):
        flat = cbase + (h[:, None, :] * WP + w[None, :, :])
        flat = jnp.where(valid, flat, ZERO_ROW).reshape(NBINS)
        return jnp.concatenate([flat, pad])

    return jnp.concatenate(
        [corner(hA, wA), corner(hA, wB), corner(hB, wA), corner(hB, wB)])


def kernel(features, rois):
    fmap = jnp.pad(jnp.transpose(features, (0, 2, 3, 1)),
                   ((0, 0), (0, 0), (0, WP - W), (0, 0)))
    idx = _roi_corner_idx(rois)

    table_u32 = pl.pallas_call(
        _table_body,
        in_specs=[pl.BlockSpec((B, H, WP, C), lambda: (0, 0, 0, 0))],
        out_specs=pl.BlockSpec((NCOMBO * B + 1, H, WP, C // 2), lambda: (0, 0, 0, 0)),
        out_shape=jax.ShapeDtypeStruct((NCOMBO * B + 1, H, WP, C // 2), jnp.uint32),
    )(fmap).reshape((NCOMBO * B + 1) * HWP, C // 2)

    mesh = plsc.VectorSubcoreMesh(core_axis_name="c", subcore_axis_name="s")
    cp = pltpu.CompilerParams()
    if "needs_layout_passes" in pltpu.CompilerParams.__dataclass_fields__:
        cp = dataclasses.replace(cp, needs_layout_passes=False)
    sc_gather = pl.kernel(
        _sc_body,
        out_type=jax.ShapeDtypeStruct((BINSP, C // 2), jnp.uint32),
        mesh=mesh,
        compiler_params=cp,
        scratch_types=[
            pltpu.VMEM((2, 4, CB), jnp.int32),
            pltpu.VMEM((2, 4, CB, C // 2), jnp.uint32),
            pltpu.VMEM((CB, C // 2), jnp.uint32),
            pltpu.SemaphoreType.DMA,
            pltpu.SemaphoreType.DMA,
        ],
    )
    binmax = sc_gather(table_u32, idx)

    out = pl.pallas_call(
        _unpack_body,
        grid=(POOL * POOL,),
        in_specs=[pl.BlockSpec((N, C // 2), lambda s: (s, 0))],
        out_specs=pl.BlockSpec((1, N, C), lambda s: (s, 0, 0)),
        out_shape=jax.ShapeDtypeStruct((POOL * POOL, N, C), jnp.float32),
    )(binmax)
    return jnp.transpose(out.reshape(POOL, POOL, N, C), (2, 3, 0, 1))


# speedup vs baseline: 1.2694x; 1.0661x over previous
import dataclasses

import jax
import jax.numpy as jnp
from jax import lax
from jax.experimental import pallas as pl
from jax.experimental.pallas import tpu as pltpu
from jax.experimental.pallas import tpu_sc as plsc

POOL = 7
SCALE = 0.0625
B, C, H, W = 2, 256, 38, 38
N = 1000
NEG = jnp.finfo(jnp.float32).min

WP = 40
HWP = H * WP
NCOMBO = 9
ZERO_ROW = NCOMBO * B * HWP

NBINS = N * POOL * POOL
NWORKERS = 32
CB = 96
BINS_PER_W = 1536
BINSP = NWORKERS * BINS_PER_W
NCHUNK = BINS_PER_W // CB


def _shift0(x, s, size):
    return jnp.concatenate([x[s:], jnp.broadcast_to(x[size - 1:], (s,) + x.shape[1:])], axis=0)


def _shift1(x, s, size):
    last = x[:, size - 1:]
    return jnp.concatenate([x[:, s:], jnp.broadcast_to(last, x.shape[:1] + (s,) + x.shape[2:])], axis=1)


def _pack_bf16_pair(x):
    u_lo = jax.lax.bitcast_convert_type(x[..., 0:C // 2], jnp.uint32)
    u_hi = jax.lax.bitcast_convert_type(x[..., C // 2:C], jnp.uint32)
    rne = lambda u: u + jnp.uint32(0x7FFF) + ((u >> 16) & jnp.uint32(1))
    lo16 = rne(u_lo) >> 16
    hi16 = rne(u_hi) & jnp.uint32(0xFFFF0000)
    return lo16 | hi16


def _table_body(fmap_ref, out_ref):
    for bb in range(B):
        f = fmap_ref[bb]
        th = f
        for kh in range(3):
            if kh:
                th = jnp.maximum(th, _shift0(th, 1 << (kh - 1), H))
            tw = th
            for kw in range(3):
                if kw:
                    tw = jnp.maximum(tw, _shift1(tw, 1 << (kw - 1), WP))
                out_ref[(kh * 3 + kw) * B + bb] = _pack_bf16_pair(tw)
    out_ref[NCOMBO * B] = jnp.zeros((H, WP, C // 2), jnp.uint32)


def _sc_body(table_hbm, idx_hbm, out_hbm, idx_v, rows_v, out_v, sem0, sem1):
    wid = lax.axis_index("s") * 2 + lax.axis_index("c")
    base = wid * BINS_PER_W

    def start_gather(t, buf, sem):
        cps = []
        for k in range(4):
            pltpu.sync_copy(idx_hbm.at[pl.ds(k * BINSP + base + t * CB, CB)],
                            idx_v.at[buf, k])
            cps.append(pltpu.async_copy(table_hbm.at[idx_v.at[buf, k]],
                                        rows_v.at[buf, k], sem))
        return cps

    def compute(buf):
        def bin_body(i):
            for j in range(C // 32):
                sl = pl.ds(j * 16, 16)
                r = [plsc.bitcast(rows_v[buf, k, i, sl], jnp.bfloat16)
                     for k in range(4)]
                m = jnp.maximum(jnp.maximum(r[0], r[1]), jnp.maximum(r[2], r[3]))
                out_v[i, sl] = plsc.bitcast(m, jnp.int32)
        pl.loop(0, CB)(bin_body)

    for cp in start_gather(0, 0, sem0):
        cp.wait()

    def chunk_body(t):
        @pl.when(t % 2 == 0)
        def _():
            cps = start_gather(t + 1, 1, sem1)
            compute(0)
            for cp in cps:
                cp.wait()

        @pl.when(t % 2 == 1)
        def _():
            cps = start_gather(t + 1, 0, sem0)
            compute(1)
            for cp in cps:
                cp.wait()

        pltpu.sync_copy(out_v, out_hbm.at[pl.ds(base + t * CB, CB)])

    pl.loop(0, NCHUNK - 1)(chunk_body)

    @pl.when((NCHUNK - 1) % 2 == 0)
    def _():
        compute(0)

    @pl.when((NCHUNK - 1) % 2 == 1)
    def _():
        compute(1)

    pltpu.sync_copy(out_v, out_hbm.at[pl.ds(base + (NCHUNK - 1) * CB, CB)])


def _unpack_body(in_ref, out_ref):
    x = in_ref[...]
    lo = jax.lax.bitcast_convert_type(x << 16, jnp.float32)
    hi = jax.lax.bitcast_convert_type(x & jnp.uint32(0xFFFF0000), jnp.float32)
    out_ref[0] = jnp.concatenate([lo, hi], axis=1)


def _roi_corner_idx(rois):
    b = rois[:, 0].astype(jnp.int32)
    rs_w = jnp.round(rois[:, 1] * SCALE).astype(jnp.int32)
    rs_h = jnp.round(rois[:, 2] * SCALE).astype(jnp.int32)
    re_w = jnp.round(rois[:, 3] * SCALE).astype(jnp.int32)
    re_h = jnp.round(rois[:, 4] * SCALE).astype(jnp.int32)
    roi_w = jnp.maximum(re_w - rs_w + 1, 1).astype(jnp.float32)
    roi_h = jnp.maximum(re_h - rs_h + 1, 1).astype(jnp.float32)
    bin_w = roi_w / POOL
    bin_h = roi_h / POOL
    p = jnp.arange(POOL, dtype=jnp.float32)[:, None]
    hstart = jnp.clip(jnp.floor(p * bin_h[None, :]).astype(jnp.int32) + rs_h[None, :], 0, H)
    hend = jnp.clip(jnp.ceil((p + 1.0) * bin_h[None, :]).astype(jnp.int32) + rs_h[None, :], 0, H)
    wstart = jnp.clip(jnp.floor(p * bin_w[None, :]).astype(jnp.int32) + rs_w[None, :], 0, W)
    wend = jnp.clip(jnp.ceil((p + 1.0) * bin_w[None, :]).astype(jnp.int32) + rs_w[None, :], 0, W)

    len_h = hend - hstart
    len_w = wend - wstart
    kh = (len_h >= 2).astype(jnp.int32) + (len_h >= 4).astype(jnp.int32)
    kw = (len_w >= 2).astype(jnp.int32) + (len_w >= 4).astype(jnp.int32)
    hA = jnp.clip(hstart, 0, H - 1)
    hB = jnp.clip(hend - (1 << kh), 0, H - 1)
    wA = jnp.clip(wstart, 0, W - 1)
    wB = jnp.clip(wend - (1 << kw), 0, W - 1)

    combo = (kh[:, None, :] * 3 + kw[None, :, :]) * B + b[None, None, :]
    cbase = combo * HWP
    valid = (len_h > 0)[:, None, :] & (len_w > 0)[None, :, :]
    pad = jnp.full((BINSP - NBINS,), ZERO_ROW, jnp.int32)

    def corner(h, w):
        flat = cbase + (h[:, None, :] * WP + w[None, :, :])
        flat = jnp.where(valid, flat, ZERO_ROW).reshape(NBINS)
        return jnp.concatenate([flat, pad])

    return jnp.concatenate(
        [corner(hA, wA), corner(hA, wB), corner(hB, wA), corner(hB, wB)])


def kernel(features, rois):
    fmap = jnp.pad(jnp.transpose(features, (0, 2, 3, 1)),
                   ((0, 0), (0, 0), (0, WP - W), (0, 0)))
    idx = _roi_corner_idx(rois)

    table_u32 = pl.pallas_call(
        _table_body,
        in_specs=[pl.BlockSpec((B, H, WP, C), lambda: (0, 0, 0, 0))],
        out_specs=pl.BlockSpec((NCOMBO * B + 1, H, WP, C // 2), lambda: (0, 0, 0, 0)),
        out_shape=jax.ShapeDtypeStruct((NCOMBO * B + 1, H, WP, C // 2), jnp.uint32),
    )(fmap).reshape((NCOMBO * B + 1) * HWP, C // 2)

    mesh = plsc.VectorSubcoreMesh(core_axis_name="c", subcore_axis_name="s")
    cp = pltpu.CompilerParams()
    if "needs_layout_passes" in pltpu.CompilerParams.__dataclass_fields__:
        cp = dataclasses.replace(cp, needs_layout_passes=False)
    sc_gather = pl.kernel(
        _sc_body,
        out_type=jax.ShapeDtypeStruct((BINSP, C // 2), jnp.uint32),
        mesh=mesh,
        compiler_params=cp,
        scratch_types=[
            pltpu.VMEM((2, 4, CB), jnp.int32),
            pltpu.VMEM((2, 4, CB, C // 2), jnp.uint32),
            pltpu.VMEM((CB, C // 2), jnp.uint32),
            pltpu.SemaphoreType.DMA,
            pltpu.SemaphoreType.DMA,
        ],
    )
    binmax = sc_gather(table_u32, idx)

    out = pl.pallas_call(
        _unpack_body,
        grid=(POOL * POOL,),
        in_specs=[pl.BlockSpec((N, C // 2), lambda s: (s, 0))],
        out_specs=pl.BlockSpec((1, N, C), lambda s: (s, 0, 0)),
        out_shape=jax.ShapeDtypeStruct((POOL * POOL, N, C), jnp.float32),
    )(binmax)
    return jnp.transpose(out.reshape(POOL, POOL, N, C), (2, 3, 0, 1))


# speedup vs baseline: 1.3095x; 1.0315x over previous
import dataclasses

import jax
import jax.numpy as jnp
from jax import lax
from jax.experimental import pallas as pl
from jax.experimental.pallas import tpu as pltpu
from jax.experimental.pallas import tpu_sc as plsc

POOL = 7
SCALE = 0.0625
B, C, H, W = 2, 256, 38, 38
N = 1000
NEG = jnp.finfo(jnp.float32).min

WP = 40
HWP = H * WP
NCOMBO = 9
ZERO_ROW = NCOMBO * B * HWP

NBINS = N * POOL * POOL
NWORKERS = 32
CB = 96
BINS_PER_W = 1536
BINSP = NWORKERS * BINS_PER_W
NCHUNK = BINS_PER_W // CB


def _shift0(x, s, size):
    return jnp.concatenate([x[s:], jnp.broadcast_to(x[size - 1:], (s,) + x.shape[1:])], axis=0)


def _shift1(x, s, size):
    last = x[:, size - 1:]
    return jnp.concatenate([x[:, s:], jnp.broadcast_to(last, x.shape[:1] + (s,) + x.shape[2:])], axis=1)


def _pack_bf16_pair(x):
    u_lo = jax.lax.bitcast_convert_type(x[..., 0:C // 2], jnp.uint32)
    u_hi = jax.lax.bitcast_convert_type(x[..., C // 2:C], jnp.uint32)
    rne = lambda u: u + jnp.uint32(0x7FFF) + ((u >> 16) & jnp.uint32(1))
    lo16 = rne(u_lo) >> 16
    hi16 = rne(u_hi) & jnp.uint32(0xFFFF0000)
    return lo16 | hi16


def _table_body(fmap_ref, out_ref):
    for bb in range(B):
        f = fmap_ref[bb]
        th = f
        for kh in range(3):
            if kh:
                th = jnp.maximum(th, _shift0(th, 1 << (kh - 1), H))
            tw = th
            for kw in range(3):
                if kw:
                    tw = jnp.maximum(tw, _shift1(tw, 1 << (kw - 1), WP))
                out_ref[(kh * 3 + kw) * B + bb] = _pack_bf16_pair(tw)
    out_ref[NCOMBO * B] = jnp.zeros((H, WP, C // 2), jnp.uint32)


def _sc_body(table_hbm, i0_hbm, i1_hbm, i2_hbm, i3_hbm, out_hbm,
             idx_v, rows_v, out_v, sem0, sem1):
    wid = lax.axis_index("s") * 2 + lax.axis_index("c")
    base = wid * BINS_PER_W
    idx_hbms = [i0_hbm, i1_hbm, i2_hbm, i3_hbm]

    def start_gather(t, buf, sem):
        cps = []
        for k in range(4):
            pltpu.sync_copy(idx_hbms[k].at[pl.ds(base + t * CB, CB)],
                            idx_v.at[buf, k])
            cps.append(pltpu.async_copy(table_hbm.at[idx_v.at[buf, k]],
                                        rows_v.at[buf, k], sem))
        return cps

    def compute(buf):
        def bin_body(i):
            for j in range(C // 32):
                sl = pl.ds(j * 16, 16)
                r = [plsc.bitcast(rows_v[buf, k, i, sl], jnp.bfloat16)
                     for k in range(4)]
                m = jnp.maximum(jnp.maximum(r[0], r[1]), jnp.maximum(r[2], r[3]))
                out_v[i, sl] = plsc.bitcast(m, jnp.int32)
        pl.loop(0, CB)(bin_body)

    for cp in start_gather(0, 0, sem0):
        cp.wait()

    def chunk_body(t):
        @pl.when(t % 2 == 0)
        def _():
            cps = start_gather(t + 1, 1, sem1)
            compute(0)
            for cp in cps:
                cp.wait()

        @pl.when(t % 2 == 1)
        def _():
            cps = start_gather(t + 1, 0, sem0)
            compute(1)
            for cp in cps:
                cp.wait()

        pltpu.sync_copy(out_v, out_hbm.at[pl.ds(base + t * CB, CB)])

    pl.loop(0, NCHUNK - 1)(chunk_body)

    @pl.when((NCHUNK - 1) % 2 == 0)
    def _():
        compute(0)

    @pl.when((NCHUNK - 1) % 2 == 1)
    def _():
        compute(1)

    pltpu.sync_copy(out_v, out_hbm.at[pl.ds(base + (NCHUNK - 1) * CB, CB)])


def _unpack_body(in_ref, out_ref):
    x = in_ref[...]
    lo = jax.lax.bitcast_convert_type(x << 16, jnp.float32)
    hi = jax.lax.bitcast_convert_type(x & jnp.uint32(0xFFFF0000), jnp.float32)
    out_ref[0] = jnp.concatenate([lo, hi], axis=1)


def _roi_corner_idx(rois):
    b = rois[:, 0].astype(jnp.int32)
    rs_w = jnp.round(rois[:, 1] * SCALE).astype(jnp.int32)
    rs_h = jnp.round(rois[:, 2] * SCALE).astype(jnp.int32)
    re_w = jnp.round(rois[:, 3] * SCALE).astype(jnp.int32)
    re_h = jnp.round(rois[:, 4] * SCALE).astype(jnp.int32)
    roi_w = jnp.maximum(re_w - rs_w + 1, 1).astype(jnp.float32)
    roi_h = jnp.maximum(re_h - rs_h + 1, 1).astype(jnp.float32)
    bin_w = roi_w / POOL
    bin_h = roi_h / POOL
    p = jnp.arange(POOL, dtype=jnp.float32)[:, None]
    hstart = jnp.clip(jnp.floor(p * bin_h[None, :]).astype(jnp.int32) + rs_h[None, :], 0, H)
    hend = jnp.clip(jnp.ceil((p + 1.0) * bin_h[None, :]).astype(jnp.int32) + rs_h[None, :], 0, H)
    wstart = jnp.clip(jnp.floor(p * bin_w[None, :]).astype(jnp.int32) + rs_w[None, :], 0, W)
    wend = jnp.clip(jnp.ceil((p + 1.0) * bin_w[None, :]).astype(jnp.int32) + rs_w[None, :], 0, W)

    len_h = hend - hstart
    len_w = wend - wstart
    kh = (len_h >= 2).astype(jnp.int32) + (len_h >= 4).astype(jnp.int32)
    kw = (len_w >= 2).astype(jnp.int32) + (len_w >= 4).astype(jnp.int32)
    hA = jnp.clip(hstart, 0, H - 1)
    hB = jnp.clip(hend - (1 << kh), 0, H - 1)
    wA = jnp.clip(wstart, 0, W - 1)
    wB = jnp.clip(wend - (1 << kw), 0, W - 1)

    combo = (kh[:, None, :] * 3 + kw[None, :, :]) * B + b[None, None, :]
    cbase = combo * HWP
    valid = (len_h > 0)[:, None, :] & (len_w > 0)[None, :, :]
    pad = jnp.full((BINSP - NBINS,), ZERO_ROW, jnp.int32)

    def corner(h, w):
        flat = cbase + (h[:, None, :] * WP + w[None, :, :])
        flat = jnp.where(valid, flat, ZERO_ROW).reshape(NBINS)
        return jnp.concatenate([flat, pad])

    return (corner(hA, wA), corner(hA, wB), corner(hB, wA), corner(hB, wB))


def kernel(features, rois):
    fmap = jnp.pad(jnp.transpose(features, (0, 2, 3, 1)),
                   ((0, 0), (0, 0), (0, WP - W), (0, 0)))
    idx4 = _roi_corner_idx(rois)

    table_u32 = pl.pallas_call(
        _table_body,
        in_specs=[pl.BlockSpec((B, H, WP, C), lambda: (0, 0, 0, 0))],
        out_specs=pl.BlockSpec((NCOMBO * B + 1, H, WP, C // 2), lambda: (0, 0, 0, 0)),
        out_shape=jax.ShapeDtypeStruct((NCOMBO * B + 1, H, WP, C // 2), jnp.uint32),
    )(fmap).reshape((NCOMBO * B + 1) * HWP, C // 2)

    mesh = plsc.VectorSubcoreMesh(core_axis_name="c", subcore_axis_name="s")
    cp = pltpu.CompilerParams()
    if "needs_layout_passes" in pltpu.CompilerParams.__dataclass_fields__:
        cp = dataclasses.replace(cp, needs_layout_passes=False)
    sc_gather = pl.kernel(
        _sc_body,
        out_type=jax.ShapeDtypeStruct((BINSP, C // 2), jnp.uint32),
        mesh=mesh,
        compiler_params=cp,
        scratch_types=[
            pltpu.VMEM((2, 4, CB), jnp.int32),
            pltpu.VMEM((2, 4, CB, C // 2), jnp.uint32),
            pltpu.VMEM((CB, C // 2), jnp.uint32),
            pltpu.SemaphoreType.DMA,
            pltpu.SemaphoreType.DMA,
        ],
    )
    binmax = sc_gather(table_u32, *idx4)

    out = pl.pallas_call(
        _unpack_body,
        grid=(POOL * POOL,),
        in_specs=[pl.BlockSpec((N, C // 2), lambda s: (s, 0))],
        out_specs=pl.BlockSpec((1, N, C), lambda s: (s, 0, 0)),
        out_shape=jax.ShapeDtypeStruct((POOL * POOL, N, C), jnp.float32),
    )(binmax)
    return jnp.transpose(out.reshape(POOL, POOL, N, C), (2, 3, 0, 1))


# speedup vs baseline: 1.5302x; 1.1686x over previous
import dataclasses

import jax
import jax.numpy as jnp
from jax import lax
from jax.experimental import pallas as pl
from jax.experimental.pallas import tpu as pltpu
from jax.experimental.pallas import tpu_sc as plsc

POOL = 7
SCALE = 0.0625
B, C, H, W = 2, 256, 38, 38
N = 1000
NEG = jnp.finfo(jnp.float32).min

WP = 40
HWP = H * WP
LH = 8
NCOMBO = LH * 3
ZERO_ROW = NCOMBO * B * HWP

NBINS = N * POOL * POOL
NWORKERS = 32
CB = 128
BINS_PER_W = 1536
BINSP = NWORKERS * BINS_PER_W
NCHUNK = BINS_PER_W // CB


def _shift0(x, s, size):
    return jnp.concatenate([x[s:], jnp.broadcast_to(x[size - 1:], (s,) + x.shape[1:])], axis=0)


def _shift1(x, s, size):
    last = x[:, size - 1:]
    return jnp.concatenate([x[:, s:], jnp.broadcast_to(last, x.shape[:1] + (s,) + x.shape[2:])], axis=1)


def _pack_bf16_pair(x):
    u_lo = jax.lax.bitcast_convert_type(x[..., 0:C // 2], jnp.uint32)
    u_hi = jax.lax.bitcast_convert_type(x[..., C // 2:C], jnp.uint32)
    rne = lambda u: u + jnp.uint32(0x7FFF) + ((u >> 16) & jnp.uint32(1))
    lo16 = rne(u_lo) >> 16
    hi16 = rne(u_hi) & jnp.uint32(0xFFFF0000)
    return lo16 | hi16


def _table_body(fmap_ref, out_ref):
    for bb in range(B):
        f = fmap_ref[bb]
        th = f
        for lh in range(1, LH + 1):
            if lh > 1:
                th = jnp.maximum(th, _shift0(f, lh - 1, H))
            tw = th
            for kw in range(3):
                if kw:
                    tw = jnp.maximum(tw, _shift1(tw, 1 << (kw - 1), WP))
                out_ref[((lh - 1) * 3 + kw) * B + bb] = _pack_bf16_pair(tw)
    out_ref[NCOMBO * B] = jnp.zeros((H, WP, C // 2), jnp.uint32)


def _sc_body(table_hbm, i0_hbm, i1_hbm, out_hbm,
             idx_v, rows_v, out_v, sem0, sem1):
    wid = lax.axis_index("s") * 2 + lax.axis_index("c")
    base = wid * BINS_PER_W
    idx_hbms = [i0_hbm, i1_hbm]

    def start_gather(t, buf, sem):
        cps = []
        for k in range(2):
            pltpu.sync_copy(idx_hbms[k].at[pl.ds(base + t * CB, CB)],
                            idx_v.at[buf, k])
            cps.append(pltpu.async_copy(table_hbm.at[idx_v.at[buf, k]],
                                        rows_v.at[buf, k], sem))
        return cps

    def compute(buf):
        def bin_body(i):
            for j in range(C // 32):
                sl = pl.ds(j * 16, 16)
                m = jnp.maximum(
                    plsc.bitcast(rows_v[buf, 0, i, sl], jnp.bfloat16),
                    plsc.bitcast(rows_v[buf, 1, i, sl], jnp.bfloat16))
                out_v[i, sl] = plsc.bitcast(m, jnp.int32)
        pl.loop(0, CB)(bin_body)

    for cp in start_gather(0, 0, sem0):
        cp.wait()

    def chunk_body(t):
        @pl.when(t % 2 == 0)
        def _():
            cps = start_gather(t + 1, 1, sem1)
            compute(0)
            for cp in cps:
                cp.wait()

        @pl.when(t % 2 == 1)
        def _():
            cps = start_gather(t + 1, 0, sem0)
            compute(1)
            for cp in cps:
                cp.wait()

        pltpu.sync_copy(out_v, out_hbm.at[pl.ds(base + t * CB, CB)])

    pl.loop(0, NCHUNK - 1)(chunk_body)

    @pl.when((NCHUNK - 1) % 2 == 0)
    def _():
        compute(0)

    @pl.when((NCHUNK - 1) % 2 == 1)
    def _():
        compute(1)

    pltpu.sync_copy(out_v, out_hbm.at[pl.ds(base + (NCHUNK - 1) * CB, CB)])


def _unpack_body(in_ref, out_ref):
    x = in_ref[...]
    lo = jax.lax.bitcast_convert_type(x << 16, jnp.float32)
    hi = jax.lax.bitcast_convert_type(x & jnp.uint32(0xFFFF0000), jnp.float32)
    out_ref[0] = jnp.concatenate([lo, hi], axis=1)


def _roi_corner_idx(rois):
    b = rois[:, 0].astype(jnp.int32)
    rs_w = jnp.round(rois[:, 1] * SCALE).astype(jnp.int32)
    rs_h = jnp.round(rois[:, 2] * SCALE).astype(jnp.int32)
    re_w = jnp.round(rois[:, 3] * SCALE).astype(jnp.int32)
    re_h = jnp.round(rois[:, 4] * SCALE).astype(jnp.int32)
    roi_w = jnp.maximum(re_w - rs_w + 1, 1).astype(jnp.float32)
    roi_h = jnp.maximum(re_h - rs_h + 1, 1).astype(jnp.float32)
    bin_w = roi_w / POOL
    bin_h = roi_h / POOL
    p = jnp.arange(POOL, dtype=jnp.float32)[:, None]
    hstart = jnp.clip(jnp.floor(p * bin_h[None, :]).astype(jnp.int32) + rs_h[None, :], 0, H)
    hend = jnp.clip(jnp.ceil((p + 1.0) * bin_h[None, :]).astype(jnp.int32) + rs_h[None, :], 0, H)
    wstart = jnp.clip(jnp.floor(p * bin_w[None, :]).astype(jnp.int32) + rs_w[None, :], 0, W)
    wend = jnp.clip(jnp.ceil((p + 1.0) * bin_w[None, :]).astype(jnp.int32) + rs_w[None, :], 0, W)

    len_h = hend - hstart
    len_w = wend - wstart
    lh = jnp.clip(len_h, 1, LH)
    kw = (len_w >= 2).astype(jnp.int32) + (len_w >= 4).astype(jnp.int32)
    hA = jnp.clip(hstart, 0, H - 1)
    wA = jnp.clip(wstart, 0, W - 1)
    wB = jnp.clip(wend - (1 << kw), 0, W - 1)

    combo = ((lh[:, None, :] - 1) * 3 + kw[None, :, :]) * B + b[None, None, :]
    cbase = combo * HWP
    valid = (len_h > 0)[:, None, :] & (len_w > 0)[None, :, :]
    pad = jnp.full((BINSP - NBINS,), ZERO_ROW, jnp.int32)

    def corner(h, w):
        flat = cbase + (h[:, None, :] * WP + w[None, :, :])
        flat = jnp.where(valid, flat, ZERO_ROW).reshape(NBINS)
        return jnp.concatenate([flat, pad])

    return (corner(hA, wA), corner(hA, wB))


def kernel(features, rois):
    fmap = jnp.pad(jnp.transpose(features, (0, 2, 3, 1)),
                   ((0, 0), (0, 0), (0, WP - W), (0, 0)))
    idx4 = _roi_corner_idx(rois)

    table_u32 = pl.pallas_call(
        _table_body,
        in_specs=[pl.BlockSpec((B, H, WP, C), lambda: (0, 0, 0, 0))],
        out_specs=pl.BlockSpec((NCOMBO * B + 1, H, WP, C // 2), lambda: (0, 0, 0, 0)),
        out_shape=jax.ShapeDtypeStruct((NCOMBO * B + 1, H, WP, C // 2), jnp.uint32),
    )(fmap).reshape((NCOMBO * B + 1) * HWP, C // 2)

    mesh = plsc.VectorSubcoreMesh(core_axis_name="c", subcore_axis_name="s")
    cp = pltpu.CompilerParams()
    if "needs_layout_passes" in pltpu.CompilerParams.__dataclass_fields__:
        cp = dataclasses.replace(cp, needs_layout_passes=False)
    sc_gather = pl.kernel(
        _sc_body,
        out_type=jax.ShapeDtypeStruct((BINSP, C // 2), jnp.uint32),
        mesh=mesh,
        compiler_params=cp,
        scratch_types=[
            pltpu.VMEM((2, 2, CB), jnp.int32),
            pltpu.VMEM((2, 2, CB, C // 2), jnp.uint32),
            pltpu.VMEM((CB, C // 2), jnp.uint32),
            pltpu.SemaphoreType.DMA,
            pltpu.SemaphoreType.DMA,
        ],
    )
    binmax = sc_gather(table_u32, *idx4)

    out = pl.pallas_call(
        _unpack_body,
        grid=(POOL * POOL,),
        in_specs=[pl.BlockSpec((N, C // 2), lambda s: (s, 0))],
        out_specs=pl.BlockSpec((1, N, C), lambda s: (s, 0, 0)),
        out_shape=jax.ShapeDtypeStruct((POOL * POOL, N, C), jnp.float32),
    )(binmax)
    return jnp.transpose(out.reshape(POOL, POOL, N, C), (2, 3, 0, 1))


# speedup vs baseline: 1.7329x; 1.1324x over previous
import dataclasses

import jax
import jax.numpy as jnp
from jax import lax
from jax.experimental import pallas as pl
from jax.experimental.pallas import tpu as pltpu
from jax.experimental.pallas import tpu_sc as plsc

POOL = 7
SCALE = 0.0625
B, C, H, W = 2, 256, 38, 38
N = 1000
NEG = jnp.finfo(jnp.float32).min

WP = 40
HWP = H * WP
LH = 8
NCOMBO = LH * 3
ZERO_ROW = NCOMBO * B * HWP

NBINS = N * POOL * POOL
NWORKERS = 32
CB = 128
BINS_PER_W = 1536
BINSP = NWORKERS * BINS_PER_W
NCHUNK = BINS_PER_W // CB


def _shift0(x, s, size):
    return jnp.concatenate([x[s:], jnp.broadcast_to(x[size - 1:], (s,) + x.shape[1:])], axis=0)


def _shift1(x, s, size):
    last = x[:, size - 1:]
    return jnp.concatenate([x[:, s:], jnp.broadcast_to(last, x.shape[:1] + (s,) + x.shape[2:])], axis=1)


def _pack_bf16_pair(x):
    u_lo = jax.lax.bitcast_convert_type(x[..., 0:C // 2], jnp.uint32)
    u_hi = jax.lax.bitcast_convert_type(x[..., C // 2:C], jnp.uint32)
    rne = lambda u: u + jnp.uint32(0x7FFF) + ((u >> 16) & jnp.uint32(1))
    lo16 = rne(u_lo) >> 16
    hi16 = rne(u_hi) & jnp.uint32(0xFFFF0000)
    return lo16 | hi16


def _table_body(fmap_ref, out_ref):
    for bb in range(B):
        f = fmap_ref[bb]
        th = f
        for lh in range(1, LH + 1):
            if lh > 1:
                th = jnp.maximum(th, _shift0(f, lh - 1, H))
            tw = th
            for kw in range(3):
                if kw:
                    tw = jnp.maximum(tw, _shift1(tw, 1 << (kw - 1), WP))
                out_ref[((lh - 1) * 3 + kw) * B + bb] = _pack_bf16_pair(tw)
    out_ref[NCOMBO * B] = jnp.zeros((H, WP, C // 2), jnp.uint32)


def _sc_body(table_hbm, i0_hbm, i1_hbm, out_hbm,
             idx_v, rows_v, out_v, sem0, sem1):
    wid = lax.axis_index("s") * 2 + lax.axis_index("c")
    base = wid * BINS_PER_W
    idx_hbms = [i0_hbm, i1_hbm]

    def start_gather(t, buf, sem):
        cps = []
        for k in range(2):
            pltpu.sync_copy(idx_hbms[k].at[pl.ds(base + t * CB, CB)],
                            idx_v.at[buf, k])
            cps.append(pltpu.async_copy(table_hbm.at[idx_v.at[buf, k]],
                                        rows_v.at[buf, k], sem))
        return cps

    def compute(buf):
        def bin_body(i):
            for j in range(C // 32):
                sl = pl.ds(j * 16, 16)
                m = jnp.maximum(
                    plsc.bitcast(rows_v[buf, 0, i, sl], jnp.bfloat16),
                    plsc.bitcast(rows_v[buf, 1, i, sl], jnp.bfloat16))
                out_v[i, sl] = plsc.bitcast(m, jnp.int32)
        pl.loop(0, CB)(bin_body)

    for cp in start_gather(0, 0, sem0):
        cp.wait()

    def chunk_body(t):
        @pl.when(t % 2 == 0)
        def _():
            cps = start_gather(t + 1, 1, sem1)
            compute(0)
            for cp in cps:
                cp.wait()

        @pl.when(t % 2 == 1)
        def _():
            cps = start_gather(t + 1, 0, sem0)
            compute(1)
            for cp in cps:
                cp.wait()

        pltpu.sync_copy(out_v, out_hbm.at[pl.ds(base + t * CB, CB)])

    pl.loop(0, NCHUNK - 1)(chunk_body)

    @pl.when((NCHUNK - 1) % 2 == 0)
    def _():
        compute(0)

    @pl.when((NCHUNK - 1) % 2 == 1)
    def _():
        compute(1)

    pltpu.sync_copy(out_v, out_hbm.at[pl.ds(base + (NCHUNK - 1) * CB, CB)])


def _unpack_body(in_ref, out_ref):
    for s in range(POOL):
        x = in_ref[pl.ds(s * N, N), :]
        lo = jax.lax.bitcast_convert_type(x << 16, jnp.float32)
        hi = jax.lax.bitcast_convert_type(x & jnp.uint32(0xFFFF0000), jnp.float32)
        out_ref[s] = jnp.concatenate([lo, hi], axis=1)


def _roi_corner_idx(rois):
    b = rois[:, 0].astype(jnp.int32)
    rs_w = jnp.round(rois[:, 1] * SCALE).astype(jnp.int32)
    rs_h = jnp.round(rois[:, 2] * SCALE).astype(jnp.int32)
    re_w = jnp.round(rois[:, 3] * SCALE).astype(jnp.int32)
    re_h = jnp.round(rois[:, 4] * SCALE).astype(jnp.int32)
    roi_w = jnp.maximum(re_w - rs_w + 1, 1).astype(jnp.float32)
    roi_h = jnp.maximum(re_h - rs_h + 1, 1).astype(jnp.float32)
    bin_w = roi_w / POOL
    bin_h = roi_h / POOL
    p = jnp.arange(POOL, dtype=jnp.float32)[:, None]
    hstart = jnp.clip(jnp.floor(p * bin_h[None, :]).astype(jnp.int32) + rs_h[None, :], 0, H)
    hend = jnp.clip(jnp.ceil((p + 1.0) * bin_h[None, :]).astype(jnp.int32) + rs_h[None, :], 0, H)
    wstart = jnp.clip(jnp.floor(p * bin_w[None, :]).astype(jnp.int32) + rs_w[None, :], 0, W)
    wend = jnp.clip(jnp.ceil((p + 1.0) * bin_w[None, :]).astype(jnp.int32) + rs_w[None, :], 0, W)

    len_h = hend - hstart
    len_w = wend - wstart
    lh = jnp.clip(len_h, 1, LH)
    kw = (len_w >= 2).astype(jnp.int32) + (len_w >= 4).astype(jnp.int32)
    hA = jnp.clip(hstart, 0, H - 1)
    wA = jnp.clip(wstart, 0, W - 1)
    wB = jnp.clip(wend - (1 << kw), 0, W - 1)

    combo = ((lh[:, None, :] - 1) * 3 + kw[None, :, :]) * B + b[None, None, :]
    cbase = combo * HWP
    valid = (len_h > 0)[:, None, :] & (len_w > 0)[None, :, :]
    pad = jnp.full((BINSP - NBINS,), ZERO_ROW, jnp.int32)

    def corner(h, w):
        flat = cbase + (h[:, None, :] * WP + w[None, :, :])
        flat = jnp.where(valid, flat, ZERO_ROW).reshape(NBINS)
        return jnp.concatenate([flat, pad])

    return (corner(hA, wA), corner(hA, wB))


def kernel(features, rois):
    fmap = jnp.pad(jnp.transpose(features, (0, 2, 3, 1)),
                   ((0, 0), (0, 0), (0, WP - W), (0, 0)))
    idx4 = _roi_corner_idx(rois)

    table_u32 = pl.pallas_call(
        _table_body,
        in_specs=[pl.BlockSpec((B, H, WP, C), lambda: (0, 0, 0, 0))],
        out_specs=pl.BlockSpec((NCOMBO * B + 1, H, WP, C // 2), lambda: (0, 0, 0, 0)),
        out_shape=jax.ShapeDtypeStruct((NCOMBO * B + 1, H, WP, C // 2), jnp.uint32),
    )(fmap).reshape((NCOMBO * B + 1) * HWP, C // 2)

    mesh = plsc.VectorSubcoreMesh(core_axis_name="c", subcore_axis_name="s")
    cp = pltpu.CompilerParams()
    if "needs_layout_passes" in pltpu.CompilerParams.__dataclass_fields__:
        cp = dataclasses.replace(cp, needs_layout_passes=False)
    sc_gather = pl.kernel(
        _sc_body,
        out_type=jax.ShapeDtypeStruct((BINSP, C // 2), jnp.uint32),
        mesh=mesh,
        compiler_params=cp,
        scratch_types=[
            pltpu.VMEM((2, 2, CB), jnp.int32),
            pltpu.VMEM((2, 2, CB, C // 2), jnp.uint32),
            pltpu.VMEM((CB, C // 2), jnp.uint32),
            pltpu.SemaphoreType.DMA,
            pltpu.SemaphoreType.DMA,
        ],
    )
    binmax = sc_gather(table_u32, *idx4)

    out = pl.pallas_call(
        _unpack_body,
        grid=(POOL,),
        in_specs=[pl.BlockSpec((POOL * N, C // 2), lambda s: (s, 0))],
        out_specs=pl.BlockSpec((POOL, N, C), lambda s: (s, 0, 0)),
        out_shape=jax.ShapeDtypeStruct((POOL * POOL, N, C), jnp.float32),
    )(binmax)
    return jnp.transpose(out.reshape(POOL, POOL, N, C), (2, 3, 0, 1))
